# baseline (speedup 1.0000x reference)
import functools

import jax
import jax.numpy as jnp
from jax import lax
from jax.experimental import pallas as pl
from jax.experimental.pallas import tpu as pltpu
from jax.experimental.pallas import tpu_sc as plsc

N_PAT = 50000
N_DRUG = 5000
E = 500000
IN = 128
HID = 128
OUT = 64
TDIM = 32

_PB = 2000
_C = 128
_NCH = 4096
_E_PAD = _NCH * _C
_CPT = _NCH // 16
_CPT_H = _NCH // 32
_DR = N_DRUG + 8
_PR = 50048
_FCP = 32
_BT = jnp.bfloat16
_RBP = 3128


def _sc_mesh():
    return plsc.VectorSubcoreMesh(core_axis_name="c", subcore_axis_name="s")


def _ring_pipeline(tab, src_v, dst_v, rows_v, acc_s, gsem, ssem, n, ring):
    for j in range(ring):
        pltpu.async_copy(tab.at[src_v.at[j]], rows_v.at[j], gsem[j])

    def round_(i):
        for j in range(ring):
            kk = i * ring + j
            pltpu.make_async_copy(tab.at[src_v.at[kk]], rows_v.at[j],
                                  gsem[j]).wait()
            pltpu.async_copy(rows_v.at[j], acc_s.at[dst_v.at[kk]],
                             ssem[j], add=True)
        for j in range(ring):
            kk = i * ring + j

            @pl.when(kk + ring < n)
            def _():
                pltpu.make_async_copy(rows_v.at[j], acc_s.at[dst_v.at[kk]],
                                      ssem[j]).wait()
                pltpu.async_copy(tab.at[src_v.at[kk + ring]], rows_v.at[j],
                                 gsem[j])

    lax.fori_loop(0, n // ring, lambda i, z: (round_(i), z)[1], 0)
    for j in range(ring):
        kk = n - ring + j
        pltpu.make_async_copy(rows_v.at[j], acc_s.at[dst_v.at[kk]],
                              ssem[j]).wait()


def _seg_sum_pd(table, src2d, dst2d, zeros_d):
    @functools.partial(
        pl.kernel,
        out_type=jax.ShapeDtypeStruct((2, _DR, HID), _BT),
        mesh=_sc_mesh(),
        compiler_params=pltpu.CompilerParams(use_tc_tiling_on_sc=False),
        scratch_types=[
            pltpu.VMEM((_CPT_H, _C), jnp.int32),
            pltpu.VMEM((_CPT_H, _C), jnp.int32),
            pltpu.VMEM((4, _C, HID), _BT),
            pltpu.VMEM_SHARED((_DR, HID), _BT),
            pltpu.SemaphoreType.DMA,
            pltpu.SemaphoreType.DMA,
            pltpu.SemaphoreType.DMA,
            pltpu.SemaphoreType.DMA,
            pltpu.SemaphoreType.DMA,
            pltpu.SemaphoreType.DMA,
            pltpu.SemaphoreType.DMA,
            pltpu.SemaphoreType.DMA,
        ],
    )
    def k(table_h, src_h, dst_h, zeros_h, out_h, src_v, dst_v, rows_v, acc_s,
          g0, g1, g2, g3, s0, s1, s2, s3):
        c = lax.axis_index("c")
        s = lax.axis_index("s")
        base = c * (_NCH // 2) + s * _CPT_H
        pltpu.sync_copy(src_h.at[pl.ds(base, _CPT_H)], src_v)
        pltpu.sync_copy(dst_h.at[pl.ds(base, _CPT_H)], dst_v)

        @pl.when(s == 0)
        def _():
            pltpu.sync_copy(zeros_h, acc_s)

        plsc.subcore_barrier()
        _ring_pipeline(table_h, src_v, dst_v, rows_v, acc_s,
                       (g0, g1, g2, g3), (s0, s1, s2, s3), _CPT_H, 4)
        plsc.subcore_barrier()

        @pl.when(s == 0)
        def _():
            pltpu.sync_copy(acc_s, out_h.at[c])

    return k(table, src2d, dst2d, zeros_d)


def _seg_sum_dp(tablef, src2d, dst2d, zeros_p):
    @functools.partial(
        pl.kernel,
        out_type=jax.ShapeDtypeStruct((4, _PR, _FCP), _BT),
        mesh=_sc_mesh(),
        compiler_params=pltpu.CompilerParams(use_tc_tiling_on_sc=False),
        scratch_types=[
            pltpu.VMEM((_CPT_H, _C), jnp.int32),
            pltpu.VMEM((_CPT_H, _C), jnp.int32),
            pltpu.VMEM((8, _C, _FCP), _BT),
            pltpu.VMEM_SHARED((_PR, _FCP), _BT),
            pltpu.SemaphoreType.DMA,
            pltpu.SemaphoreType.DMA,
            pltpu.SemaphoreType.DMA,
            pltpu.SemaphoreType.DMA,
            pltpu.SemaphoreType.DMA,
            pltpu.SemaphoreType.DMA,
            pltpu.SemaphoreType.DMA,
            pltpu.SemaphoreType.DMA,
            pltpu.SemaphoreType.DMA,
            pltpu.SemaphoreType.DMA,
            pltpu.SemaphoreType.DMA,
            pltpu.SemaphoreType.DMA,
            pltpu.SemaphoreType.DMA,
            pltpu.SemaphoreType.DMA,
            pltpu.SemaphoreType.DMA,
            pltpu.SemaphoreType.DMA,
        ],
    )
    def k(table_h, src_h, dst_h, zeros_h, out_h, src_v, dst_v, rows_v, acc_s,
          g0, g1, g2, g3, g4, g5, g6, g7, s0, s1, s2, s3, s4, s5, s6, s7):
        c = lax.axis_index("c")
        s = lax.axis_index("s")

        for fp in range(2):
            q = c * 2 + fp
            tab = table_h.at[pl.ds(q * N_DRUG, N_DRUG)]

            @pl.when(s == 0)
            def _():
                pltpu.sync_copy(zeros_h, acc_s)

            plsc.subcore_barrier()
            for h in range(2):
                base = s * _CPT + h * _CPT_H
                pltpu.sync_copy(src_h.at[pl.ds(base, _CPT_H)], src_v)
                pltpu.sync_copy(dst_h.at[pl.ds(base, _CPT_H)], dst_v)
                _ring_pipeline(tab, src_v, dst_v, rows_v, acc_s,
                               (g0, g1, g2, g3, g4, g5, g6, g7),
                               (s0, s1, s2, s3, s4, s5, s6, s7), _CPT_H, 8)
            plsc.subcore_barrier()

            @pl.when(s == 0)
            def _():
                pltpu.sync_copy(acc_s, out_h.at[q])

            plsc.subcore_barrier()

    return k(tablef, src2d, dst2d, zeros_p)


def _seg_counts(dst_pd2d, dst_dp2d, ones, zeros_d8, zeros_p8):
    @functools.partial(
        pl.kernel,
        out_type=[jax.ShapeDtypeStruct((2, _DR, 8), jnp.float32),
                  jax.ShapeDtypeStruct((2, _PR, 8), jnp.float32)],
        mesh=_sc_mesh(),
        compiler_params=pltpu.CompilerParams(use_tc_tiling_on_sc=False),
        scratch_types=[
            pltpu.VMEM((_CPT_H, _C), jnp.int32),
            pltpu.VMEM((_CPT_H, _C), jnp.int32),
            pltpu.VMEM((_C, 8), jnp.float32),
            pltpu.VMEM_SHARED((_DR, 8), jnp.float32),
            pltpu.VMEM_SHARED((_PR, 8), jnp.float32),
            pltpu.SemaphoreType.DMA,
            pltpu.SemaphoreType.DMA,
        ],
    )
    def k(dpd_h, ddp_h, ones_h, zd_h, zp_h, outd_h, outp_h,
          dpd_v, ddp_v, ones_v, accd_s, accp_s, sd, sp):
        c = lax.axis_index("c")
        s = lax.axis_index("s")
        base = c * (_NCH // 2) + s * _CPT_H
        pltpu.sync_copy(dpd_h.at[pl.ds(base, _CPT_H)], dpd_v)
        pltpu.sync_copy(ddp_h.at[pl.ds(base, _CPT_H)], ddp_v)
        pltpu.sync_copy(ones_h, ones_v)

        @pl.when(s == 0)
        def _():
            pltpu.sync_copy(zd_h, accd_s)
            pltpu.sync_copy(zp_h, accp_s)

        plsc.subcore_barrier()

        def round_(i):
            for j in range(4):
                kk = i * 4 + j
                pltpu.async_copy(ones_v, accd_s.at[dpd_v.at[kk]], sd, add=True)
                pltpu.async_copy(ones_v, accp_s.at[ddp_v.at[kk]], sp, add=True)
            for j in range(4):
                kk = i * 4 + j
                pltpu.make_async_copy(ones_v, accd_s.at[dpd_v.at[kk]],
                                      sd).wait()
                pltpu.make_async_copy(ones_v, accp_s.at[ddp_v.at[kk]],
                                      sp).wait()

        lax.fori_loop(0, _CPT_H // 4, lambda i, z: (round_(i), z)[1], 0)
        plsc.subcore_barrier()

        @pl.when(s == 0)
        def _():
            pltpu.sync_copy(accd_s, outd_h.at[c])
            pltpu.sync_copy(accp_s, outp_h.at[c])

    return k(dst_pd2d, dst_dp2d, ones, zeros_d8, zeros_p8)


def _prologue_patient_body(xp_ref, t_ref, tlw_ref, tlb_ref, tpw_ref, tpb_ref,
                           ppw_ref, ppb_ref, win_ref, bin_ref,
                           out_ref, outb_ref):
    t = t_ref[...]
    lin = t * tlw_ref[0, 0] + tlb_ref[0]
    per = jnp.sin(t @ ppw_ref[...].T + ppb_ref[...][None, :])
    t2v = jnp.concatenate([lin, per], axis=-1)
    xp = xp_ref[...] + jnp.tanh(
        jnp.dot(t2v, tpw_ref[...].T, preferred_element_type=jnp.float32)
        + tpb_ref[...][None, :])
    y = jnp.tanh(
        jnp.dot(xp, win_ref[...].T, preferred_element_type=jnp.float32)
        + bin_ref[...][None, :])
    out_ref[...] = y
    outb_ref[...] = y.astype(_BT)


def _prologue_patient(x_patient, patient_time, t2v_lin_w, t2v_lin_b,
                      tp_w, tp_b, t2v_per_w, t2v_per_b, W_in, b_in):
    nb = N_PAT // _PB
    full = lambda *s: pl.BlockSpec(s, lambda i: tuple(0 for _ in s))
    return pl.pallas_call(
        _prologue_patient_body,
        grid=(nb,),
        in_specs=[
            pl.BlockSpec((_PB, IN), lambda i: (i, 0)),
            pl.BlockSpec((_PB, 1), lambda i: (i, 0)),
            full(1, 1), full(1), full(IN, TDIM), full(IN),
            full(TDIM - 1, 1), full(TDIM - 1), full(HID, IN), full(HID),
        ],
        out_specs=[pl.BlockSpec((_PB, HID), lambda i: (i, 0)),
                   pl.BlockSpec((_PB, HID), lambda i: (i, 0))],
        out_shape=[jax.ShapeDtypeStruct((N_PAT, HID), jnp.float32),
                   jax.ShapeDtypeStruct((N_PAT, HID), _BT)],
    )(x_patient, patient_time[:, None], t2v_lin_w, t2v_lin_b, tp_w, tp_b,
      t2v_per_w, t2v_per_b, W_in, b_in)


def _chunk_store_flat(outc_ref, y):
    yb = y.astype(_BT)
    for q in range(4):
        outc_ref[pl.ds(q * N_DRUG, N_DRUG), :] = yb[:, q * _FCP:(q + 1) * _FCP]


def _prologue_drug_body(xd_ref, dsf_ref, dsw_ref, dsb_ref, win_ref, bin_ref,
                        out_ref, outc_ref):
    xd = xd_ref[...] + jnp.tanh(
        jnp.dot(dsf_ref[...], dsw_ref[...].T, preferred_element_type=jnp.float32)
        + dsb_ref[...][None, :])
    y = jnp.tanh(
        jnp.dot(xd, win_ref[...].T, preferred_element_type=jnp.float32)
        + bin_ref[...][None, :])
    out_ref[...] = y
    _chunk_store_flat(outc_ref, y)


def _prologue_drug(x_drug, drug_struct_feat, ds_w, ds_b, W_in, b_in):
    return pl.pallas_call(
        _prologue_drug_body,
        out_shape=[jax.ShapeDtypeStruct((N_DRUG, HID), jnp.float32),
                   jax.ShapeDtypeStruct((4 * N_DRUG, _FCP), _BT)],
    )(x_drug, drug_struct_feat, ds_w, ds_b, W_in, b_in)


def _recip_body(parts_ref, out_ref):
    x = parts_ref[...]
    cnt = x[0, :, 0:1] + x[1, :, 0:1]
    out_ref[...] = 1.0 / jnp.maximum(cnt, 1.0)


def _recip_drug(parts):
    return pl.pallas_call(
        _recip_body,
        out_shape=jax.ShapeDtypeStruct((_DR, 1), jnp.float32),
    )(parts)


def _recip_patient(parts):
    nb = _PR // _RBP
    return pl.pallas_call(
        _recip_body,
        grid=(nb,),
        in_specs=[pl.BlockSpec((2, _RBP, 8), lambda i: (0, i, 0))],
        out_specs=pl.BlockSpec((_RBP, 1), lambda i: (i, 0)),
        out_shape=jax.ShapeDtypeStruct((_PR, 1), jnp.float32),
    )(parts)


def _combine_drug_body(sum_ref, recip_ref, x_ref, wl_ref, bl_ref, wr_ref,
                       out_ref, outc_ref):
    ssum = (sum_ref[0, :N_DRUG, :].astype(jnp.float32)
            + sum_ref[1, :N_DRUG, :].astype(jnp.float32))
    agg = ssum * recip_ref[:N_DRUG, :]
    y = (jnp.dot(agg, wl_ref[...].T, preferred_element_type=jnp.float32)
         + bl_ref[...][None, :]
         + jnp.dot(x_ref[...], wr_ref[...].T,
                   preferred_element_type=jnp.float32))
    out_ref[...] = y
    _chunk_store_flat(outc_ref, y)


def _combine_drug(sumd, recip, x_dst, Wl, bl, Wr):
    return pl.pallas_call(
        _combine_drug_body,
        out_shape=[jax.ShapeDtypeStruct((N_DRUG, HID), jnp.float32),
                   jax.ShapeDtypeStruct((4 * N_DRUG, _FCP), _BT)],
    )(sumd, recip, x_dst, Wl, bl, Wr)


def _combine_patient_body(sum_ref, recip_ref, x_ref, wl_ref, bl_ref, wr_ref,
                          out_ref, outb_ref):
    parts = sum_ref[...].astype(jnp.float32)
    ssum = jnp.concatenate([parts[q] for q in range(4)], axis=1)
    agg = ssum * recip_ref[...]
    y = (jnp.dot(agg, wl_ref[...].T, preferred_element_type=jnp.float32)
         + bl_ref[...][None, :]
         + jnp.dot(x_ref[...], wr_ref[...].T,
                   preferred_element_type=jnp.float32))
    out_ref[...] = y
    outb_ref[...] = y.astype(_BT)


def _combine_patient(sump, recip, x_dst, Wl, bl, Wr):
    nb = N_PAT // _PB
    full = lambda *s: pl.BlockSpec(s, lambda i: tuple(0 for _ in s))
    return pl.pallas_call(
        _combine_patient_body,
        grid=(nb,),
        in_specs=[
            pl.BlockSpec((4, _PB, _FCP), lambda i: (0, i, 0)),
            pl.BlockSpec((_PB, 1), lambda i: (i, 0)),
            pl.BlockSpec((_PB, HID), lambda i: (i, 0)),
            full(HID, HID), full(HID), full(HID, HID),
        ],
        out_specs=[pl.BlockSpec((_PB, HID), lambda i: (i, 0)),
                   pl.BlockSpec((_PB, HID), lambda i: (i, 0))],
        out_shape=[jax.ShapeDtypeStruct((N_PAT, HID), jnp.float32),
                   jax.ShapeDtypeStruct((N_PAT, HID), _BT)],
    )(sump, recip, x_dst, Wl, bl, Wr)


def _epilogue_body(xp_ref, pdsa_ref, daw_ref, dab_ref, g_ref, row_ref,
                   rob_ref, out_ref):
    g = 2.0 * jax.nn.sigmoid(g_ref[0]) - 1.0
    hidden = xp_ref[...] + g * jnp.tanh(
        jnp.dot(pdsa_ref[...], daw_ref[...].T, preferred_element_type=jnp.float32)
        + dab_ref[...][None, :])
    out_ref[...] = (
        jnp.dot(hidden, row_ref[...].T, preferred_element_type=jnp.float32)
        + rob_ref[...][None, :])


def _epilogue(xp, pdsa, da_w, da_b, gate, ro_w, ro_b):
    nb = N_PAT // _PB
    full = lambda *s: pl.BlockSpec(s, lambda i: tuple(0 for _ in s))
    return pl.pallas_call(
        _epilogue_body,
        grid=(nb,),
        in_specs=[
            pl.BlockSpec((_PB, HID), lambda i: (i, 0)),
            pl.BlockSpec((_PB, 64), lambda i: (i, 0)),
            full(HID, 64), full(HID), full(1), full(OUT, HID), full(OUT),
        ],
        out_specs=pl.BlockSpec((_PB, OUT), lambda i: (i, 0)),
        out_shape=jax.ShapeDtypeStruct((N_PAT, OUT), jnp.float32),
    )(xp, pdsa, da_w, da_b, gate, ro_w, ro_b)


def _pad2d(idx, fill):
    pad = jnp.full((_E_PAD - E,), fill, jnp.int32)
    return jnp.concatenate([idx, pad]).reshape(_NCH, _C)


def kernel(x_patient, x_drug, patient_time, drug_struct_feat,
           patient_drug_struct_agg, edge_index_patient_drug,
           edge_index_drug_patient, W_in, b_in, t2v_lin_w, t2v_lin_b,
           t2v_per_w, t2v_per_b, tp_w, tp_b, ds_w, ds_b, da_w, da_b, gate,
           s0pd_Wl, s0pd_bl, s0pd_Wr, s0dp_Wl, s0dp_bl, s0dp_Wr,
           s1pd_Wl, s1pd_bl, s1pd_Wr, s1dp_Wl, s1dp_bl, s1dp_Wr,
           ro_w, ro_b):
    src_pd = _pad2d(edge_index_patient_drug[0], 0)
    dst_pd = _pad2d(edge_index_patient_drug[1], N_DRUG)
    src_dp = _pad2d(edge_index_drug_patient[0], 0)
    dst_dp = _pad2d(edge_index_drug_patient[1], N_PAT)
    zeros_d = jnp.zeros((_DR, HID), _BT)
    zeros_p = jnp.zeros((_PR, _FCP), _BT)
    zeros_d8 = jnp.zeros((_DR, 8), jnp.float32)
    zeros_p8 = jnp.zeros((_PR, 8), jnp.float32)
    ones = jnp.ones((_C, 8), jnp.float32)

    xp, xpb = _prologue_patient(x_patient, patient_time, t2v_lin_w, t2v_lin_b,
                                tp_w, tp_b, t2v_per_w, t2v_per_b, W_in, b_in)
    xd, xdc = _prologue_drug(x_drug, drug_struct_feat, ds_w, ds_b, W_in, b_in)

    cntd_parts, cntp_parts = _seg_counts(dst_pd, dst_dp, ones,
                                         zeros_d8, zeros_p8)
    recip_d = _recip_drug(cntd_parts)
    recip_p = _recip_patient(cntp_parts)

    sumd = _seg_sum_pd(xpb, src_pd, dst_pd, zeros_d)
    sump = _seg_sum_dp(xdc, src_dp, dst_dp, zeros_p)
    xd1, xdc1 = _combine_drug(sumd, recip_d, xd, s0pd_Wl, s0pd_bl, s0pd_Wr)
    xp1, xpb1 = _combine_patient(sump, recip_p, xp, s0dp_Wl, s0dp_bl, s0dp_Wr)

    sump2 = _seg_sum_dp(xdc1, src_dp, dst_dp, zeros_p)
    xp2, _ = _combine_patient(sump2, recip_p, xp1, s1dp_Wl, s1dp_bl, s1dp_Wr)

    return _epilogue(xp2, patient_drug_struct_agg, da_w, da_b, gate, ro_w, ro_b)


# speedup vs baseline: 1.1590x; 1.1590x over previous
import functools

import jax
import jax.numpy as jnp
from jax import lax
from jax.experimental import pallas as pl
from jax.experimental.pallas import tpu as pltpu
from jax.experimental.pallas import tpu_sc as plsc

N_PAT = 50000
N_DRUG = 5000
E = 500000
IN = 128
HID = 128
OUT = 64
TDIM = 32

_PB = 2000
_C = 128
_NCH = 4096
_E_PAD = _NCH * _C
_CPT = _NCH // 16
_CPT_H = _NCH // 32
_DR = N_DRUG + 8
_PR = 50048
_FCP = 32
_BT = jnp.bfloat16
_RBP = 3128


def _sc_mesh():
    return plsc.VectorSubcoreMesh(core_axis_name="c", subcore_axis_name="s")


def _ring_pipeline(tab, src_v, dst_v, rows_v, acc_s, gsem, ssem, n, ring):
    for j in range(ring):
        pltpu.async_copy(tab.at[src_v.at[j]], rows_v.at[j], gsem[j])

    def round_(i):
        for j in range(ring):
            kk = i * ring + j
            pltpu.make_async_copy(tab.at[src_v.at[kk]], rows_v.at[j],
                                  gsem[j]).wait()
            pltpu.async_copy(rows_v.at[j], acc_s.at[dst_v.at[kk]],
                             ssem[j], add=True)
        for j in range(ring):
            kk = i * ring + j

            @pl.when(kk + ring < n)
            def _():
                pltpu.make_async_copy(rows_v.at[j], acc_s.at[dst_v.at[kk]],
                                      ssem[j]).wait()
                pltpu.async_copy(tab.at[src_v.at[kk + ring]], rows_v.at[j],
                                 gsem[j])

    lax.fori_loop(0, n // ring, lambda i, z: (round_(i), z)[1], 0)
    for j in range(ring):
        kk = n - ring + j
        pltpu.make_async_copy(rows_v.at[j], acc_s.at[dst_v.at[kk]],
                              ssem[j]).wait()


_FCD = 64


def _seg_sum_pd(tablef, src2d, dst2d, zeros_d):
    @functools.partial(
        pl.kernel,
        out_type=jax.ShapeDtypeStruct((2, _DR, _FCD), _BT),
        mesh=_sc_mesh(),
        compiler_params=pltpu.CompilerParams(use_tc_tiling_on_sc=False),
        scratch_types=[
            pltpu.VMEM((_CPT_H, _C), jnp.int32),
            pltpu.VMEM((_CPT_H, _C), jnp.int32),
            pltpu.VMEM((8, _C, _FCD), _BT),
            pltpu.VMEM_SHARED((_DR, _FCD), _BT),
            pltpu.SemaphoreType.DMA,
            pltpu.SemaphoreType.DMA,
            pltpu.SemaphoreType.DMA,
            pltpu.SemaphoreType.DMA,
            pltpu.SemaphoreType.DMA,
            pltpu.SemaphoreType.DMA,
            pltpu.SemaphoreType.DMA,
            pltpu.SemaphoreType.DMA,
            pltpu.SemaphoreType.DMA,
            pltpu.SemaphoreType.DMA,
            pltpu.SemaphoreType.DMA,
            pltpu.SemaphoreType.DMA,
            pltpu.SemaphoreType.DMA,
            pltpu.SemaphoreType.DMA,
            pltpu.SemaphoreType.DMA,
            pltpu.SemaphoreType.DMA,
        ],
    )
    def k(table_h, src_h, dst_h, zeros_h, out_h, src_v, dst_v, rows_v, acc_s,
          g0, g1, g2, g3, g4, g5, g6, g7, s0, s1, s2, s3, s4, s5, s6, s7):
        c = lax.axis_index("c")
        s = lax.axis_index("s")
        tab = table_h.at[c]

        @pl.when(s == 0)
        def _():
            pltpu.sync_copy(zeros_h, acc_s)

        plsc.subcore_barrier()
        for h in range(2):
            base = s * _CPT + h * _CPT_H
            pltpu.sync_copy(src_h.at[pl.ds(base, _CPT_H)], src_v)
            pltpu.sync_copy(dst_h.at[pl.ds(base, _CPT_H)], dst_v)
            _ring_pipeline(tab, src_v, dst_v, rows_v, acc_s,
                           (g0, g1, g2, g3, g4, g5, g6, g7),
                           (s0, s1, s2, s3, s4, s5, s6, s7), _CPT_H, 8)
        plsc.subcore_barrier()

        @pl.when(s == 0)
        def _():
            pltpu.sync_copy(acc_s, out_h.at[c])

    return k(tablef, src2d, dst2d, zeros_d)


def _seg_sum_dp(tablef, src2d, dst2d, zeros_p):
    @functools.partial(
        pl.kernel,
        out_type=jax.ShapeDtypeStruct((4, _PR, _FCP), _BT),
        mesh=_sc_mesh(),
        compiler_params=pltpu.CompilerParams(use_tc_tiling_on_sc=False),
        scratch_types=[
            pltpu.VMEM((_CPT_H, _C), jnp.int32),
            pltpu.VMEM((_CPT_H, _C), jnp.int32),
            pltpu.VMEM((8, _C, _FCP), _BT),
            pltpu.VMEM_SHARED((_PR, _FCP), _BT),
            pltpu.SemaphoreType.DMA,
            pltpu.SemaphoreType.DMA,
            pltpu.SemaphoreType.DMA,
            pltpu.SemaphoreType.DMA,
            pltpu.SemaphoreType.DMA,
            pltpu.SemaphoreType.DMA,
            pltpu.SemaphoreType.DMA,
            pltpu.SemaphoreType.DMA,
            pltpu.SemaphoreType.DMA,
            pltpu.SemaphoreType.DMA,
            pltpu.SemaphoreType.DMA,
            pltpu.SemaphoreType.DMA,
            pltpu.SemaphoreType.DMA,
            pltpu.SemaphoreType.DMA,
            pltpu.SemaphoreType.DMA,
            pltpu.SemaphoreType.DMA,
        ],
    )
    def k(table_h, src_h, dst_h, zeros_h, out_h, src_v, dst_v, rows_v, acc_s,
          g0, g1, g2, g3, g4, g5, g6, g7, s0, s1, s2, s3, s4, s5, s6, s7):
        c = lax.axis_index("c")
        s = lax.axis_index("s")

        for fp in range(2):
            q = c * 2 + fp
            tab = table_h.at[pl.ds(q * N_DRUG, N_DRUG)]

            @pl.when(s == 0)
            def _():
                pltpu.sync_copy(zeros_h, acc_s)

            plsc.subcore_barrier()
            for h in range(2):
                base = s * _CPT + h * _CPT_H
                pltpu.sync_copy(src_h.at[pl.ds(base, _CPT_H)], src_v)
                pltpu.sync_copy(dst_h.at[pl.ds(base, _CPT_H)], dst_v)
                _ring_pipeline(tab, src_v, dst_v, rows_v, acc_s,
                               (g0, g1, g2, g3, g4, g5, g6, g7),
                               (s0, s1, s2, s3, s4, s5, s6, s7), _CPT_H, 8)
            plsc.subcore_barrier()

            @pl.when(s == 0)
            def _():
                pltpu.sync_copy(acc_s, out_h.at[q])

            plsc.subcore_barrier()

    return k(tablef, src2d, dst2d, zeros_p)


def _seg_counts(dst_pd2d, dst_dp2d, ones, zeros_d8, zeros_p8):
    @functools.partial(
        pl.kernel,
        out_type=[jax.ShapeDtypeStruct((2, _DR, 8), jnp.float32),
                  jax.ShapeDtypeStruct((2, _PR, 8), jnp.float32)],
        mesh=_sc_mesh(),
        compiler_params=pltpu.CompilerParams(use_tc_tiling_on_sc=False),
        scratch_types=[
            pltpu.VMEM((_CPT_H, _C), jnp.int32),
            pltpu.VMEM((_CPT_H, _C), jnp.int32),
            pltpu.VMEM((_C, 8), jnp.float32),
            pltpu.VMEM_SHARED((_DR, 8), jnp.float32),
            pltpu.VMEM_SHARED((_PR, 8), jnp.float32),
            pltpu.SemaphoreType.DMA,
            pltpu.SemaphoreType.DMA,
        ],
    )
    def k(dpd_h, ddp_h, ones_h, zd_h, zp_h, outd_h, outp_h,
          dpd_v, ddp_v, ones_v, accd_s, accp_s, sd, sp):
        c = lax.axis_index("c")
        s = lax.axis_index("s")
        base = c * (_NCH // 2) + s * _CPT_H
        pltpu.sync_copy(dpd_h.at[pl.ds(base, _CPT_H)], dpd_v)
        pltpu.sync_copy(ddp_h.at[pl.ds(base, _CPT_H)], ddp_v)
        pltpu.sync_copy(ones_h, ones_v)

        @pl.when(s == 0)
        def _():
            pltpu.sync_copy(zd_h, accd_s)
            pltpu.sync_copy(zp_h, accp_s)

        plsc.subcore_barrier()

        def round_(i):
            for j in range(4):
                kk = i * 4 + j
                pltpu.async_copy(ones_v, accd_s.at[dpd_v.at[kk]], sd, add=True)
                pltpu.async_copy(ones_v, accp_s.at[ddp_v.at[kk]], sp, add=True)
            for j in range(4):
                kk = i * 4 + j
                pltpu.make_async_copy(ones_v, accd_s.at[dpd_v.at[kk]],
                                      sd).wait()
                pltpu.make_async_copy(ones_v, accp_s.at[ddp_v.at[kk]],
                                      sp).wait()

        lax.fori_loop(0, _CPT_H // 4, lambda i, z: (round_(i), z)[1], 0)
        plsc.subcore_barrier()

        @pl.when(s == 0)
        def _():
            pltpu.sync_copy(accd_s, outd_h.at[c])
            pltpu.sync_copy(accp_s, outp_h.at[c])

    return k(dst_pd2d, dst_dp2d, ones, zeros_d8, zeros_p8)


def _prologue_patient_body(xp_ref, t_ref, tlw_ref, tlb_ref, tpw_ref, tpb_ref,
                           ppw_ref, ppb_ref, win_ref, bin_ref,
                           out_ref, outb_ref):
    t = t_ref[...]
    lin = t * tlw_ref[0, 0] + tlb_ref[0]
    per = jnp.sin(t @ ppw_ref[...].T + ppb_ref[...][None, :])
    t2v = jnp.concatenate([lin, per], axis=-1)
    xp = xp_ref[...] + jnp.tanh(
        jnp.dot(t2v, tpw_ref[...].T, preferred_element_type=jnp.float32)
        + tpb_ref[...][None, :])
    y = jnp.tanh(
        jnp.dot(xp, win_ref[...].T, preferred_element_type=jnp.float32)
        + bin_ref[...][None, :])
    out_ref[...] = y
    yb = y.astype(_BT)
    outb_ref[0, :, :] = yb[:, :_FCD]
    outb_ref[1, :, :] = yb[:, _FCD:]


def _prologue_patient(x_patient, patient_time, t2v_lin_w, t2v_lin_b,
                      tp_w, tp_b, t2v_per_w, t2v_per_b, W_in, b_in):
    nb = N_PAT // _PB
    full = lambda *s: pl.BlockSpec(s, lambda i: tuple(0 for _ in s))
    return pl.pallas_call(
        _prologue_patient_body,
        grid=(nb,),
        in_specs=[
            pl.BlockSpec((_PB, IN), lambda i: (i, 0)),
            pl.BlockSpec((_PB, 1), lambda i: (i, 0)),
            full(1, 1), full(1), full(IN, TDIM), full(IN),
            full(TDIM - 1, 1), full(TDIM - 1), full(HID, IN), full(HID),
        ],
        out_specs=[pl.BlockSpec((_PB, HID), lambda i: (i, 0)),
                   pl.BlockSpec((2, _PB, _FCD), lambda i: (0, i, 0))],
        out_shape=[jax.ShapeDtypeStruct((N_PAT, HID), jnp.float32),
                   jax.ShapeDtypeStruct((2, N_PAT, _FCD), _BT)],
    )(x_patient, patient_time[:, None], t2v_lin_w, t2v_lin_b, tp_w, tp_b,
      t2v_per_w, t2v_per_b, W_in, b_in)


def _chunk_store_flat(outc_ref, y):
    yb = y.astype(_BT)
    for q in range(4):
        outc_ref[pl.ds(q * N_DRUG, N_DRUG), :] = yb[:, q * _FCP:(q + 1) * _FCP]


def _prologue_drug_body(xd_ref, dsf_ref, dsw_ref, dsb_ref, win_ref, bin_ref,
                        out_ref, outc_ref):
    xd = xd_ref[...] + jnp.tanh(
        jnp.dot(dsf_ref[...], dsw_ref[...].T, preferred_element_type=jnp.float32)
        + dsb_ref[...][None, :])
    y = jnp.tanh(
        jnp.dot(xd, win_ref[...].T, preferred_element_type=jnp.float32)
        + bin_ref[...][None, :])
    out_ref[...] = y
    _chunk_store_flat(outc_ref, y)


def _prologue_drug(x_drug, drug_struct_feat, ds_w, ds_b, W_in, b_in):
    return pl.pallas_call(
        _prologue_drug_body,
        out_shape=[jax.ShapeDtypeStruct((N_DRUG, HID), jnp.float32),
                   jax.ShapeDtypeStruct((4 * N_DRUG, _FCP), _BT)],
    )(x_drug, drug_struct_feat, ds_w, ds_b, W_in, b_in)


def _recip_body(parts_ref, out_ref):
    x = parts_ref[...]
    cnt = x[0, :, 0:1] + x[1, :, 0:1]
    out_ref[...] = 1.0 / jnp.maximum(cnt, 1.0)


def _recip_drug(parts):
    return pl.pallas_call(
        _recip_body,
        out_shape=jax.ShapeDtypeStruct((_DR, 1), jnp.float32),
    )(parts)


def _recip_patient(parts):
    nb = _PR // _RBP
    return pl.pallas_call(
        _recip_body,
        grid=(nb,),
        in_specs=[pl.BlockSpec((2, _RBP, 8), lambda i: (0, i, 0))],
        out_specs=pl.BlockSpec((_RBP, 1), lambda i: (i, 0)),
        out_shape=jax.ShapeDtypeStruct((_PR, 1), jnp.float32),
    )(parts)


def _combine_drug_body(sum_ref, recip_ref, x_ref, wl_ref, bl_ref, wr_ref,
                       outc_ref):
    parts = sum_ref[...].astype(jnp.float32)
    ssum = jnp.concatenate([parts[0], parts[1]], axis=1)[:N_DRUG, :]
    agg = ssum * recip_ref[:N_DRUG, :]
    y = (jnp.dot(agg, wl_ref[...].T, preferred_element_type=jnp.float32)
         + bl_ref[...][None, :]
         + jnp.dot(x_ref[...], wr_ref[...].T,
                   preferred_element_type=jnp.float32))
    _chunk_store_flat(outc_ref, y)


def _combine_drug(sumd, recip, x_dst, Wl, bl, Wr):
    return pl.pallas_call(
        _combine_drug_body,
        out_shape=jax.ShapeDtypeStruct((4 * N_DRUG, _FCP), _BT),
    )(sumd, recip, x_dst, Wl, bl, Wr)


def _combine_patient_body(sum_ref, recip_ref, x_ref, wl_ref, bl_ref, wr_ref,
                          out_ref):
    parts = sum_ref[...].astype(jnp.float32)
    ssum = jnp.concatenate([parts[q] for q in range(4)], axis=1)
    agg = ssum * recip_ref[...]
    out_ref[...] = (
        jnp.dot(agg, wl_ref[...].T, preferred_element_type=jnp.float32)
        + bl_ref[...][None, :]
        + jnp.dot(x_ref[...], wr_ref[...].T,
                  preferred_element_type=jnp.float32))


def _combine_patient(sump, recip, x_dst, Wl, bl, Wr):
    nb = N_PAT // _PB
    full = lambda *s: pl.BlockSpec(s, lambda i: tuple(0 for _ in s))
    return pl.pallas_call(
        _combine_patient_body,
        grid=(nb,),
        in_specs=[
            pl.BlockSpec((4, _PB, _FCP), lambda i: (0, i, 0)),
            pl.BlockSpec((_PB, 1), lambda i: (i, 0)),
            pl.BlockSpec((_PB, HID), lambda i: (i, 0)),
            full(HID, HID), full(HID), full(HID, HID),
        ],
        out_specs=pl.BlockSpec((_PB, HID), lambda i: (i, 0)),
        out_shape=jax.ShapeDtypeStruct((N_PAT, HID), jnp.float32),
    )(sump, recip, x_dst, Wl, bl, Wr)


def _epilogue_body(xp_ref, pdsa_ref, daw_ref, dab_ref, g_ref, row_ref,
                   rob_ref, out_ref):
    g = 2.0 * jax.nn.sigmoid(g_ref[0]) - 1.0
    hidden = xp_ref[...] + g * jnp.tanh(
        jnp.dot(pdsa_ref[...], daw_ref[...].T, preferred_element_type=jnp.float32)
        + dab_ref[...][None, :])
    out_ref[...] = (
        jnp.dot(hidden, row_ref[...].T, preferred_element_type=jnp.float32)
        + rob_ref[...][None, :])


def _epilogue(xp, pdsa, da_w, da_b, gate, ro_w, ro_b):
    nb = N_PAT // _PB
    full = lambda *s: pl.BlockSpec(s, lambda i: tuple(0 for _ in s))
    return pl.pallas_call(
        _epilogue_body,
        grid=(nb,),
        in_specs=[
            pl.BlockSpec((_PB, HID), lambda i: (i, 0)),
            pl.BlockSpec((_PB, 64), lambda i: (i, 0)),
            full(HID, 64), full(HID), full(1), full(OUT, HID), full(OUT),
        ],
        out_specs=pl.BlockSpec((_PB, OUT), lambda i: (i, 0)),
        out_shape=jax.ShapeDtypeStruct((N_PAT, OUT), jnp.float32),
    )(xp, pdsa, da_w, da_b, gate, ro_w, ro_b)


def _pad2d(idx, fill):
    pad = jnp.full((_E_PAD - E,), fill, jnp.int32)
    return jnp.concatenate([idx, pad]).reshape(_NCH, _C)


def kernel(x_patient, x_drug, patient_time, drug_struct_feat,
           patient_drug_struct_agg, edge_index_patient_drug,
           edge_index_drug_patient, W_in, b_in, t2v_lin_w, t2v_lin_b,
           t2v_per_w, t2v_per_b, tp_w, tp_b, ds_w, ds_b, da_w, da_b, gate,
           s0pd_Wl, s0pd_bl, s0pd_Wr, s0dp_Wl, s0dp_bl, s0dp_Wr,
           s1pd_Wl, s1pd_bl, s1pd_Wr, s1dp_Wl, s1dp_bl, s1dp_Wr,
           ro_w, ro_b):
    src_pd = _pad2d(edge_index_patient_drug[0], 0)
    dst_pd = _pad2d(edge_index_patient_drug[1], N_DRUG)
    src_dp = _pad2d(edge_index_drug_patient[0], 0)
    dst_dp = _pad2d(edge_index_drug_patient[1], N_PAT)
    zeros_d = jnp.zeros((_DR, _FCD), _BT)
    zeros_p = jnp.zeros((_PR, _FCP), _BT)
    zeros_d8 = jnp.zeros((_DR, 8), jnp.float32)
    zeros_p8 = jnp.zeros((_PR, 8), jnp.float32)
    ones = jnp.ones((_C, 8), jnp.float32)

    xp, xpb = _prologue_patient(x_patient, patient_time, t2v_lin_w, t2v_lin_b,
                                tp_w, tp_b, t2v_per_w, t2v_per_b, W_in, b_in)
    xd, xdc = _prologue_drug(x_drug, drug_struct_feat, ds_w, ds_b, W_in, b_in)

    cntd_parts, cntp_parts = _seg_counts(dst_pd, dst_dp, ones,
                                         zeros_d8, zeros_p8)
    recip_d = _recip_drug(cntd_parts)
    recip_p = _recip_patient(cntp_parts)

    sumd = _seg_sum_pd(xpb, src_pd, dst_pd, zeros_d)
    sump = _seg_sum_dp(xdc, src_dp, dst_dp, zeros_p)
    xdc1 = _combine_drug(sumd, recip_d, xd, s0pd_Wl, s0pd_bl, s0pd_Wr)
    xp1 = _combine_patient(sump, recip_p, xp, s0dp_Wl, s0dp_bl, s0dp_Wr)

    sump2 = _seg_sum_dp(xdc1, src_dp, dst_dp, zeros_p)
    xp2 = _combine_patient(sump2, recip_p, xp1, s1dp_Wl, s1dp_bl, s1dp_Wr)

    return _epilogue(xp2, patient_drug_struct_agg, da_w, da_b, gate, ro_w, ro_b)


# speedup vs baseline: 1.1600x; 1.0009x over previous
import functools

import jax
import jax.numpy as jnp
from jax import lax
from jax.experimental import pallas as pl
from jax.experimental.pallas import tpu as pltpu
from jax.experimental.pallas import tpu_sc as plsc

N_PAT = 50000
N_DRUG = 5000
E = 500000
IN = 128
HID = 128
OUT = 64
TDIM = 32

_PB = 5000
_C = 128
_NCH = 4096
_E_PAD = _NCH * _C
_CPT = _NCH // 16
_CPT_H = _NCH // 32
_DR = N_DRUG + 8
_PR = 50048
_FCP = 32
_BT = jnp.bfloat16
_RBP = 3128


def _sc_mesh():
    return plsc.VectorSubcoreMesh(core_axis_name="c", subcore_axis_name="s")


def _ring_pipeline(tab, src_v, dst_v, rows_v, acc_s, gsem, ssem, n, ring):
    for j in range(ring):
        pltpu.async_copy(tab.at[src_v.at[j]], rows_v.at[j], gsem[j])

    def round_(i):
        for j in range(ring):
            kk = i * ring + j
            pltpu.make_async_copy(tab.at[src_v.at[kk]], rows_v.at[j],
                                  gsem[j]).wait()
            pltpu.async_copy(rows_v.at[j], acc_s.at[dst_v.at[kk]],
                             ssem[j], add=True)
        for j in range(ring):
            kk = i * ring + j

            @pl.when(kk + ring < n)
            def _():
                pltpu.make_async_copy(rows_v.at[j], acc_s.at[dst_v.at[kk]],
                                      ssem[j]).wait()
                pltpu.async_copy(tab.at[src_v.at[kk + ring]], rows_v.at[j],
                                 gsem[j])

    lax.fori_loop(0, n // ring, lambda i, z: (round_(i), z)[1], 0)
    for j in range(ring):
        kk = n - ring + j
        pltpu.make_async_copy(rows_v.at[j], acc_s.at[dst_v.at[kk]],
                              ssem[j]).wait()


_FCD = 64


def _seg_sum_pd(tablef, src2d, dst2d, zeros_d):
    @functools.partial(
        pl.kernel,
        out_type=jax.ShapeDtypeStruct((2, _DR, _FCD), _BT),
        mesh=_sc_mesh(),
        compiler_params=pltpu.CompilerParams(use_tc_tiling_on_sc=False),
        scratch_types=[
            pltpu.VMEM((_CPT_H, _C), jnp.int32),
            pltpu.VMEM((_CPT_H, _C), jnp.int32),
            pltpu.VMEM((8, _C, _FCD), _BT),
            pltpu.VMEM_SHARED((_DR, _FCD), _BT),
            pltpu.SemaphoreType.DMA,
            pltpu.SemaphoreType.DMA,
            pltpu.SemaphoreType.DMA,
            pltpu.SemaphoreType.DMA,
            pltpu.SemaphoreType.DMA,
            pltpu.SemaphoreType.DMA,
            pltpu.SemaphoreType.DMA,
            pltpu.SemaphoreType.DMA,
            pltpu.SemaphoreType.DMA,
            pltpu.SemaphoreType.DMA,
            pltpu.SemaphoreType.DMA,
            pltpu.SemaphoreType.DMA,
            pltpu.SemaphoreType.DMA,
            pltpu.SemaphoreType.DMA,
            pltpu.SemaphoreType.DMA,
            pltpu.SemaphoreType.DMA,
        ],
    )
    def k(table_h, src_h, dst_h, zeros_h, out_h, src_v, dst_v, rows_v, acc_s,
          g0, g1, g2, g3, g4, g5, g6, g7, s0, s1, s2, s3, s4, s5, s6, s7):
        c = lax.axis_index("c")
        s = lax.axis_index("s")
        tab = table_h.at[c]

        @pl.when(s == 0)
        def _():
            pltpu.sync_copy(zeros_h, acc_s)

        plsc.subcore_barrier()
        for h in range(2):
            base = s * _CPT + h * _CPT_H
            pltpu.sync_copy(src_h.at[pl.ds(base, _CPT_H)], src_v)
            pltpu.sync_copy(dst_h.at[pl.ds(base, _CPT_H)], dst_v)
            _ring_pipeline(tab, src_v, dst_v, rows_v, acc_s,
                           (g0, g1, g2, g3, g4, g5, g6, g7),
                           (s0, s1, s2, s3, s4, s5, s6, s7), _CPT_H, 8)
        plsc.subcore_barrier()

        @pl.when(s == 0)
        def _():
            pltpu.sync_copy(acc_s, out_h.at[c])

    return k(tablef, src2d, dst2d, zeros_d)


def _seg_sum_dp(tablef, src2d, dst2d, zeros_p):
    @functools.partial(
        pl.kernel,
        out_type=jax.ShapeDtypeStruct((4, _PR, _FCP), _BT),
        mesh=_sc_mesh(),
        compiler_params=pltpu.CompilerParams(use_tc_tiling_on_sc=False),
        scratch_types=[
            pltpu.VMEM((_CPT_H, _C), jnp.int32),
            pltpu.VMEM((_CPT_H, _C), jnp.int32),
            pltpu.VMEM((8, _C, _FCP), _BT),
            pltpu.VMEM_SHARED((_PR, _FCP), _BT),
            pltpu.SemaphoreType.DMA,
            pltpu.SemaphoreType.DMA,
            pltpu.SemaphoreType.DMA,
            pltpu.SemaphoreType.DMA,
            pltpu.SemaphoreType.DMA,
            pltpu.SemaphoreType.DMA,
            pltpu.SemaphoreType.DMA,
            pltpu.SemaphoreType.DMA,
            pltpu.SemaphoreType.DMA,
            pltpu.SemaphoreType.DMA,
            pltpu.SemaphoreType.DMA,
            pltpu.SemaphoreType.DMA,
            pltpu.SemaphoreType.DMA,
            pltpu.SemaphoreType.DMA,
            pltpu.SemaphoreType.DMA,
            pltpu.SemaphoreType.DMA,
        ],
    )
    def k(table_h, src_h, dst_h, zeros_h, out_h, src_v, dst_v, rows_v, acc_s,
          g0, g1, g2, g3, g4, g5, g6, g7, s0, s1, s2, s3, s4, s5, s6, s7):
        c = lax.axis_index("c")
        s = lax.axis_index("s")

        for fp in range(2):
            q = c * 2 + fp
            tab = table_h.at[pl.ds(q * N_DRUG, N_DRUG)]

            @pl.when(s == 0)
            def _():
                pltpu.sync_copy(zeros_h, acc_s)

            plsc.subcore_barrier()
            for h in range(2):
                base = s * _CPT + h * _CPT_H
                pltpu.sync_copy(src_h.at[pl.ds(base, _CPT_H)], src_v)
                pltpu.sync_copy(dst_h.at[pl.ds(base, _CPT_H)], dst_v)
                _ring_pipeline(tab, src_v, dst_v, rows_v, acc_s,
                               (g0, g1, g2, g3, g4, g5, g6, g7),
                               (s0, s1, s2, s3, s4, s5, s6, s7), _CPT_H, 8)
            plsc.subcore_barrier()

            @pl.when(s == 0)
            def _():
                pltpu.sync_copy(acc_s, out_h.at[q])

            plsc.subcore_barrier()

    return k(tablef, src2d, dst2d, zeros_p)


def _seg_counts(dst_pd2d, dst_dp2d, ones, zeros_d8, zeros_p8):
    @functools.partial(
        pl.kernel,
        out_type=[jax.ShapeDtypeStruct((2, _DR, 8), jnp.float32),
                  jax.ShapeDtypeStruct((2, _PR, 8), jnp.float32)],
        mesh=_sc_mesh(),
        compiler_params=pltpu.CompilerParams(use_tc_tiling_on_sc=False),
        scratch_types=[
            pltpu.VMEM((_CPT_H, _C), jnp.int32),
            pltpu.VMEM((_CPT_H, _C), jnp.int32),
            pltpu.VMEM((_C, 8), jnp.float32),
            pltpu.VMEM_SHARED((_DR, 8), jnp.float32),
            pltpu.VMEM_SHARED((_PR, 8), jnp.float32),
            pltpu.SemaphoreType.DMA,
            pltpu.SemaphoreType.DMA,
        ],
    )
    def k(dpd_h, ddp_h, ones_h, zd_h, zp_h, outd_h, outp_h,
          dpd_v, ddp_v, ones_v, accd_s, accp_s, sd, sp):
        c = lax.axis_index("c")
        s = lax.axis_index("s")
        base = c * (_NCH // 2) + s * _CPT_H
        pltpu.sync_copy(dpd_h.at[pl.ds(base, _CPT_H)], dpd_v)
        pltpu.sync_copy(ddp_h.at[pl.ds(base, _CPT_H)], ddp_v)
        pltpu.sync_copy(ones_h, ones_v)

        @pl.when(s == 0)
        def _():
            pltpu.sync_copy(zd_h, accd_s)
            pltpu.sync_copy(zp_h, accp_s)

        plsc.subcore_barrier()

        def round_(i):
            for j in range(4):
                kk = i * 4 + j
                pltpu.async_copy(ones_v, accd_s.at[dpd_v.at[kk]], sd, add=True)
                pltpu.async_copy(ones_v, accp_s.at[ddp_v.at[kk]], sp, add=True)
            for j in range(4):
                kk = i * 4 + j
                pltpu.make_async_copy(ones_v, accd_s.at[dpd_v.at[kk]],
                                      sd).wait()
                pltpu.make_async_copy(ones_v, accp_s.at[ddp_v.at[kk]],
                                      sp).wait()

        lax.fori_loop(0, _CPT_H // 4, lambda i, z: (round_(i), z)[1], 0)
        plsc.subcore_barrier()

        @pl.when(s == 0)
        def _():
            pltpu.sync_copy(accd_s, outd_h.at[c])
            pltpu.sync_copy(accp_s, outp_h.at[c])

    return k(dst_pd2d, dst_dp2d, ones, zeros_d8, zeros_p8)


def _prologue_patient_body(xp_ref, t_ref, tlw_ref, tlb_ref, tpw_ref, tpb_ref,
                           ppw_ref, ppb_ref, win_ref, bin_ref,
                           out_ref, outb_ref):
    t = t_ref[...]
    lin = t * tlw_ref[0, 0] + tlb_ref[0]
    per = jnp.sin(t @ ppw_ref[...].T + ppb_ref[...][None, :])
    t2v = jnp.concatenate([lin, per], axis=-1)
    xp = xp_ref[...] + jnp.tanh(
        jnp.dot(t2v, tpw_ref[...].T, preferred_element_type=jnp.float32)
        + tpb_ref[...][None, :])
    y = jnp.tanh(
        jnp.dot(xp, win_ref[...].T, preferred_element_type=jnp.float32)
        + bin_ref[...][None, :])
    out_ref[...] = y
    yb = y.astype(_BT)
    outb_ref[0, :, :] = yb[:, :_FCD]
    outb_ref[1, :, :] = yb[:, _FCD:]


def _prologue_patient(x_patient, patient_time, t2v_lin_w, t2v_lin_b,
                      tp_w, tp_b, t2v_per_w, t2v_per_b, W_in, b_in):
    nb = N_PAT // _PB
    full = lambda *s: pl.BlockSpec(s, lambda i: tuple(0 for _ in s))
    return pl.pallas_call(
        _prologue_patient_body,
        grid=(nb,),
        in_specs=[
            pl.BlockSpec((_PB, IN), lambda i: (i, 0)),
            pl.BlockSpec((_PB, 1), lambda i: (i, 0)),
            full(1, 1), full(1), full(IN, TDIM), full(IN),
            full(TDIM - 1, 1), full(TDIM - 1), full(HID, IN), full(HID),
        ],
        out_specs=[pl.BlockSpec((_PB, HID), lambda i: (i, 0)),
                   pl.BlockSpec((2, _PB, _FCD), lambda i: (0, i, 0))],
        out_shape=[jax.ShapeDtypeStruct((N_PAT, HID), jnp.float32),
                   jax.ShapeDtypeStruct((2, N_PAT, _FCD), _BT)],
    )(x_patient, patient_time[:, None], t2v_lin_w, t2v_lin_b, tp_w, tp_b,
      t2v_per_w, t2v_per_b, W_in, b_in)


def _chunk_store_flat(outc_ref, y):
    yb = y.astype(_BT)
    for q in range(4):
        outc_ref[pl.ds(q * N_DRUG, N_DRUG), :] = yb[:, q * _FCP:(q + 1) * _FCP]


def _prologue_drug_body(xd_ref, dsf_ref, dsw_ref, dsb_ref, win_ref, bin_ref,
                        out_ref, outc_ref):
    xd = xd_ref[...] + jnp.tanh(
        jnp.dot(dsf_ref[...], dsw_ref[...].T, preferred_element_type=jnp.float32)
        + dsb_ref[...][None, :])
    y = jnp.tanh(
        jnp.dot(xd, win_ref[...].T, preferred_element_type=jnp.float32)
        + bin_ref[...][None, :])
    out_ref[...] = y
    _chunk_store_flat(outc_ref, y)


def _prologue_drug(x_drug, drug_struct_feat, ds_w, ds_b, W_in, b_in):
    return pl.pallas_call(
        _prologue_drug_body,
        out_shape=[jax.ShapeDtypeStruct((N_DRUG, HID), jnp.float32),
                   jax.ShapeDtypeStruct((4 * N_DRUG, _FCP), _BT)],
    )(x_drug, drug_struct_feat, ds_w, ds_b, W_in, b_in)


def _recip_body(parts_ref, out_ref):
    x = parts_ref[...]
    cnt = x[0, :, 0:1] + x[1, :, 0:1]
    out_ref[...] = 1.0 / jnp.maximum(cnt, 1.0)


def _recip_drug(parts):
    return pl.pallas_call(
        _recip_body,
        out_shape=jax.ShapeDtypeStruct((_DR, 1), jnp.float32),
    )(parts)


def _recip_patient(parts):
    nb = _PR // _RBP
    return pl.pallas_call(
        _recip_body,
        grid=(nb,),
        in_specs=[pl.BlockSpec((2, _RBP, 8), lambda i: (0, i, 0))],
        out_specs=pl.BlockSpec((_RBP, 1), lambda i: (i, 0)),
        out_shape=jax.ShapeDtypeStruct((_PR, 1), jnp.float32),
    )(parts)


def _combine_drug_body(sum_ref, recip_ref, x_ref, wl_ref, bl_ref, wr_ref,
                       outc_ref):
    parts = sum_ref[...].astype(jnp.float32)
    ssum = jnp.concatenate([parts[0], parts[1]], axis=1)[:N_DRUG, :]
    agg = ssum * recip_ref[:N_DRUG, :]
    y = (jnp.dot(agg, wl_ref[...].T, preferred_element_type=jnp.float32)
         + bl_ref[...][None, :]
         + jnp.dot(x_ref[...], wr_ref[...].T,
                   preferred_element_type=jnp.float32))
    _chunk_store_flat(outc_ref, y)


def _combine_drug(sumd, recip, x_dst, Wl, bl, Wr):
    return pl.pallas_call(
        _combine_drug_body,
        out_shape=jax.ShapeDtypeStruct((4 * N_DRUG, _FCP), _BT),
    )(sumd, recip, x_dst, Wl, bl, Wr)


def _combine_patient_body(sum_ref, recip_ref, x_ref, wl_ref, bl_ref, wr_ref,
                          out_ref):
    parts = sum_ref[...].astype(jnp.float32)
    ssum = jnp.concatenate([parts[q] for q in range(4)], axis=1)
    agg = ssum * recip_ref[...]
    out_ref[...] = (
        jnp.dot(agg, wl_ref[...].T, preferred_element_type=jnp.float32)
        + bl_ref[...][None, :]
        + jnp.dot(x_ref[...], wr_ref[...].T,
                  preferred_element_type=jnp.float32))


def _combine_patient(sump, recip, x_dst, Wl, bl, Wr):
    nb = N_PAT // _PB
    full = lambda *s: pl.BlockSpec(s, lambda i: tuple(0 for _ in s))
    return pl.pallas_call(
        _combine_patient_body,
        grid=(nb,),
        in_specs=[
            pl.BlockSpec((4, _PB, _FCP), lambda i: (0, i, 0)),
            pl.BlockSpec((_PB, 1), lambda i: (i, 0)),
            pl.BlockSpec((_PB, HID), lambda i: (i, 0)),
            full(HID, HID), full(HID), full(HID, HID),
        ],
        out_specs=pl.BlockSpec((_PB, HID), lambda i: (i, 0)),
        out_shape=jax.ShapeDtypeStruct((N_PAT, HID), jnp.float32),
    )(sump, recip, x_dst, Wl, bl, Wr)


def _epilogue_body(xp_ref, pdsa_ref, daw_ref, dab_ref, g_ref, row_ref,
                   rob_ref, out_ref):
    g = 2.0 * jax.nn.sigmoid(g_ref[0]) - 1.0
    hidden = xp_ref[...] + g * jnp.tanh(
        jnp.dot(pdsa_ref[...], daw_ref[...].T, preferred_element_type=jnp.float32)
        + dab_ref[...][None, :])
    out_ref[...] = (
        jnp.dot(hidden, row_ref[...].T, preferred_element_type=jnp.float32)
        + rob_ref[...][None, :])


def _epilogue(xp, pdsa, da_w, da_b, gate, ro_w, ro_b):
    nb = N_PAT // _PB
    full = lambda *s: pl.BlockSpec(s, lambda i: tuple(0 for _ in s))
    return pl.pallas_call(
        _epilogue_body,
        grid=(nb,),
        in_specs=[
            pl.BlockSpec((_PB, HID), lambda i: (i, 0)),
            pl.BlockSpec((_PB, 64), lambda i: (i, 0)),
            full(HID, 64), full(HID), full(1), full(OUT, HID), full(OUT),
        ],
        out_specs=pl.BlockSpec((_PB, OUT), lambda i: (i, 0)),
        out_shape=jax.ShapeDtypeStruct((N_PAT, OUT), jnp.float32),
    )(xp, pdsa, da_w, da_b, gate, ro_w, ro_b)


def _pad2d(idx, fill):
    pad = jnp.full((_E_PAD - E,), fill, jnp.int32)
    return jnp.concatenate([idx, pad]).reshape(_NCH, _C)


def kernel(x_patient, x_drug, patient_time, drug_struct_feat,
           patient_drug_struct_agg, edge_index_patient_drug,
           edge_index_drug_patient, W_in, b_in, t2v_lin_w, t2v_lin_b,
           t2v_per_w, t2v_per_b, tp_w, tp_b, ds_w, ds_b, da_w, da_b, gate,
           s0pd_Wl, s0pd_bl, s0pd_Wr, s0dp_Wl, s0dp_bl, s0dp_Wr,
           s1pd_Wl, s1pd_bl, s1pd_Wr, s1dp_Wl, s1dp_bl, s1dp_Wr,
           ro_w, ro_b):
    src_pd = _pad2d(edge_index_patient_drug[0], 0)
    dst_pd = _pad2d(edge_index_patient_drug[1], N_DRUG)
    src_dp = _pad2d(edge_index_drug_patient[0], 0)
    dst_dp = _pad2d(edge_index_drug_patient[1], N_PAT)
    zeros_d = jnp.zeros((_DR, _FCD), _BT)
    zeros_p = jnp.zeros((_PR, _FCP), _BT)
    zeros_d8 = jnp.zeros((_DR, 8), jnp.float32)
    zeros_p8 = jnp.zeros((_PR, 8), jnp.float32)
    ones = jnp.ones((_C, 8), jnp.float32)

    xp, xpb = _prologue_patient(x_patient, patient_time, t2v_lin_w, t2v_lin_b,
                                tp_w, tp_b, t2v_per_w, t2v_per_b, W_in, b_in)
    xd, xdc = _prologue_drug(x_drug, drug_struct_feat, ds_w, ds_b, W_in, b_in)

    cntd_parts, cntp_parts = _seg_counts(dst_pd, dst_dp, ones,
                                         zeros_d8, zeros_p8)
    recip_d = _recip_drug(cntd_parts)
    recip_p = _recip_patient(cntp_parts)

    sumd = _seg_sum_pd(xpb, src_pd, dst_pd, zeros_d)
    sump = _seg_sum_dp(xdc, src_dp, dst_dp, zeros_p)
    xdc1 = _combine_drug(sumd, recip_d, xd, s0pd_Wl, s0pd_bl, s0pd_Wr)
    sump2 = _seg_sum_dp(xdc1, src_dp, dst_dp, zeros_p)
    xp1 = _combine_patient(sump, recip_p, xp, s0dp_Wl, s0dp_bl, s0dp_Wr)
    xp2 = _combine_patient(sump2, recip_p, xp1, s1dp_Wl, s1dp_bl, s1dp_Wr)

    return _epilogue(xp2, patient_drug_struct_agg, da_w, da_b, gate, ro_w, ro_b)
